# baseline (speedup 1.0000x reference)
import jax
import jax.numpy as jnp
from jax.experimental import pallas as pl
from jax.experimental.pallas import tpu as pltpu

_N = 4096
_BOX = 20.0
_CUTOFF = 2.5
_HIDDEN = 16
_ROWS = 512


def _energy_kernel(xyz_ref, xt_ref, w1_ref, b1_ref, w2_ref, b2_ref, out_ref):
    i = pl.program_id(0)
    half = 0.5 * _BOX

    dsq = jnp.zeros((_ROWS, _N), jnp.float32)
    for c in range(3):
        col = xt_ref[c : c + 1, :]
        row = xyz_ref[:, c : c + 1]
        d = col - row
        d = d + jnp.where(d < -half, _BOX, 0.0) - jnp.where(d >= half, _BOX, 0.0)
        dsq = dsq + d * d

    mask = (dsq < _CUTOFF * _CUTOFF) & (dsq > 0.0)
    dist = jnp.sqrt(jnp.where(mask, dsq, 1.0))

    e = jnp.full((_ROWS, _N), b2_ref[0], jnp.float32)
    for k in range(_HIDDEN):
        e = e + w2_ref[k, 0] * jnp.tanh(dist * w1_ref[0, k] + b1_ref[k])

    block_sum = jnp.sum(jnp.where(mask, e, 0.0))

    @pl.when(i == 0)
    def _init():
        out_ref[0, 0] = 0.0

    out_ref[0, 0] += block_sum


def kernel(xyz, W1, b1, W2, b2):
    xt = xyz.T
    grid = _N // _ROWS
    out = pl.pallas_call(
        _energy_kernel,
        grid=(grid,),
        in_specs=[
            pl.BlockSpec((_ROWS, 3), lambda i: (i, 0)),
            pl.BlockSpec((3, _N), lambda i: (0, 0)),
            pl.BlockSpec(memory_space=pltpu.SMEM),
            pl.BlockSpec(memory_space=pltpu.SMEM),
            pl.BlockSpec(memory_space=pltpu.SMEM),
            pl.BlockSpec(memory_space=pltpu.SMEM),
        ],
        out_specs=pl.BlockSpec(memory_space=pltpu.SMEM),
        out_shape=jax.ShapeDtypeStruct((1, 1), jnp.float32),
    )(xyz, xt, W1, b1, W2, b2)
    return out[0, 0]


# speedup vs baseline: 1.0263x; 1.0263x over previous
import jax
import jax.numpy as jnp
from jax.experimental import pallas as pl
from jax.experimental.pallas import tpu as pltpu

_N = 4096
_BOX = 20.0
_CUTOFF = 2.5
_HIDDEN = 16
_ROWS = 512


def _energy_kernel(xyz_ref, xt_ref, w1_ref, b1_ref, w2_ref, b2_ref, out_ref):
    i = pl.program_id(0)

    dsq = jnp.zeros((_ROWS, _N), jnp.float32)
    for c in range(3):
        col = xt_ref[c : c + 1, :]
        row = xyz_ref[:, c : c + 1]
        d = col - row
        d = d - _BOX * jnp.round(d * (1.0 / _BOX))
        dsq = dsq + d * d

    mask = (dsq < _CUTOFF * _CUTOFF) & (dsq > 0.0)
    dist = jnp.sqrt(jnp.where(mask, dsq, 1.0))

    e0 = jnp.full((_ROWS, _N), b2_ref[0], jnp.float32)
    e1 = jnp.zeros((_ROWS, _N), jnp.float32)
    for k in range(0, _HIDDEN, 2):
        e0 = e0 + w2_ref[k, 0] * jnp.tanh(dist * w1_ref[0, k] + b1_ref[k])
        e1 = e1 + w2_ref[k + 1, 0] * jnp.tanh(dist * w1_ref[0, k + 1] + b1_ref[k + 1])

    block_sum = jnp.sum(jnp.where(mask, e0 + e1, 0.0))

    @pl.when(i == 0)
    def _init():
        out_ref[0, 0] = 0.0

    out_ref[0, 0] += block_sum


def kernel(xyz, W1, b1, W2, b2):
    xt = xyz.T
    grid = _N // _ROWS
    out = pl.pallas_call(
        _energy_kernel,
        grid=(grid,),
        in_specs=[
            pl.BlockSpec((_ROWS, 3), lambda i: (i, 0)),
            pl.BlockSpec((3, _N), lambda i: (0, 0)),
            pl.BlockSpec(memory_space=pltpu.SMEM),
            pl.BlockSpec(memory_space=pltpu.SMEM),
            pl.BlockSpec(memory_space=pltpu.SMEM),
            pl.BlockSpec(memory_space=pltpu.SMEM),
        ],
        out_specs=pl.BlockSpec(memory_space=pltpu.SMEM),
        out_shape=jax.ShapeDtypeStruct((1, 1), jnp.float32),
    )(xyz, xt, W1, b1, W2, b2)
    return out[0, 0]


# speedup vs baseline: 3.1960x; 3.1142x over previous
import jax
import jax.numpy as jnp
from jax.experimental import pallas as pl
from jax.experimental.pallas import tpu as pltpu

_N = 4096
_BOX = 20.0
_CUTOFF = 2.5
_HIDDEN = 16
_ROWS = 256
_COLT = 256
_NRB = _N // _ROWS
_NCT = _N // _COLT


def _energy_kernel(starts_ref, ntiles_ref, xyz_ref, xt_ref,
                   w1_ref, b1_ref, w2_ref, b2_ref, out_ref):
    i = pl.program_id(0)
    t = pl.program_id(1)

    @pl.when(jnp.logical_and(i == 0, t == 0))
    def _init():
        out_ref[0, 0] = 0.0

    @pl.when(t < ntiles_ref[i])
    def _body():
        dsq = jnp.zeros((_ROWS, _COLT), jnp.float32)
        for c in range(3):
            col = xt_ref[c : c + 1, :]
            row = xyz_ref[:, c : c + 1]
            d = col - row
            d = d - _BOX * jnp.round(d * (1.0 / _BOX))
            dsq = dsq + d * d

        mask = (dsq < _CUTOFF * _CUTOFF) & (dsq > 0.0)
        dist = jnp.sqrt(jnp.where(mask, dsq, 1.0))

        e0 = jnp.full((_ROWS, _COLT), b2_ref[0], jnp.float32)
        e1 = jnp.zeros((_ROWS, _COLT), jnp.float32)
        for k in range(0, _HIDDEN, 2):
            e0 = e0 + w2_ref[k, 0] * jnp.tanh(dist * w1_ref[0, k] + b1_ref[k])
            e1 = e1 + w2_ref[k + 1, 0] * jnp.tanh(dist * w1_ref[0, k + 1] + b1_ref[k + 1])

        out_ref[0, 0] += jnp.sum(jnp.where(mask, e0 + e1, 0.0))


def _col_index(i, t, starts_ref, ntiles_ref):
    tt = jnp.minimum(t, ntiles_ref[i] - 1)
    return (0, (starts_ref[i] + tt) % _NCT)


def kernel(xyz, W1, b1, W2, b2):
    order = jnp.argsort(xyz[:, 0])
    xyzs = xyz[order]
    xs = xyzs[:, 0]

    xb = xs.reshape(_NRB, _ROWS)
    lo_val = xb[:, 0] - _CUTOFF
    hi_val = xb[:, -1] + _CUTOFF
    full = (hi_val - lo_val) >= _BOX
    lo_m = jnp.mod(lo_val, _BOX)
    hi_m = jnp.mod(hi_val, _BOX)
    lo_idx = jnp.searchsorted(xs, lo_m, side="left").astype(jnp.int32)
    hi_idx = jnp.searchsorted(xs, hi_m, side="right").astype(jnp.int32)
    start_tile = lo_idx // _COLT
    end_tile = (hi_idx + _COLT - 1) // _COLT
    n_lin = end_tile - start_tile
    n_wrap = _NCT - start_tile + end_tile
    n_tiles = jnp.where(hi_m >= lo_m, n_lin, n_wrap)
    n_tiles = jnp.where(full, _NCT, n_tiles)
    n_tiles = jnp.clip(n_tiles, 1, _NCT).astype(jnp.int32)
    start_tile = start_tile.astype(jnp.int32)

    grid_spec = pltpu.PrefetchScalarGridSpec(
        num_scalar_prefetch=2,
        grid=(_NRB, _NCT),
        in_specs=[
            pl.BlockSpec((_ROWS, 3), lambda i, t, s, n: (i, 0)),
            pl.BlockSpec((3, _COLT), _col_index),
            pl.BlockSpec(memory_space=pltpu.SMEM),
            pl.BlockSpec(memory_space=pltpu.SMEM),
            pl.BlockSpec(memory_space=pltpu.SMEM),
            pl.BlockSpec(memory_space=pltpu.SMEM),
        ],
        out_specs=pl.BlockSpec(memory_space=pltpu.SMEM),
    )
    out = pl.pallas_call(
        _energy_kernel,
        grid_spec=grid_spec,
        out_shape=jax.ShapeDtypeStruct((1, 1), jnp.float32),
    )(start_tile, n_tiles, xyzs, xyzs.T, W1, b1, W2, b2)
    return out[0, 0]
